# baseline (speedup 1.0000x reference)
import functools

import jax
import jax.numpy as jnp
from jax import lax
from jax.experimental import pallas as pl
from jax.experimental.pallas import tpu as pltpu
from jax.experimental.pallas import tpu_sc as plsc

N = 10000
D = 128
H = 4
C = 32
NH = 16
NC = 2
NS = 16
HW = 64
B = 128
E0 = 320000
EP = 331776
EPT = EP // NS
NB = EPT // B
NPAD = 10016
ROWS_PT = NPAD // NS
AW = 80
def _proj_body(x_ref, w_ref, wd_ref, wl_ref, wr_ref, xl0_ref, xl1_ref,
               xr0_ref, xr1_ref):
    xb = x_ref[...]
    ms = jnp.mean(xb * xb, axis=1, keepdims=True)
    hb = xb * lax.rsqrt(ms + 1e-6) * w_ref[...]
    hb = jnp.dot(hb, wd_ref[...], preferred_element_type=jnp.float32)
    xl = jnp.dot(hb, wl_ref[...], preferred_element_type=jnp.float32)
    xr = jnp.dot(hb, wr_ref[...], preferred_element_type=jnp.float32)
    xl0_ref[...] = xl[:, :HW]
    xl1_ref[...] = xl[:, HW:]
    xr0_ref[...] = xr[:, :HW]
    xr1_ref[...] = xr[:, HW:]


def _projections(x, rms_weight, W_down, W_l, W_r):
    blk = 1000
    return pl.pallas_call(
        _proj_body,
        grid=(N // blk,),
        in_specs=[
            pl.BlockSpec((blk, D), lambda i: (i, 0)),
            pl.BlockSpec((1, D), lambda i: (0, 0)),
            pl.BlockSpec((D, D), lambda i: (0, 0)),
            pl.BlockSpec((D, D), lambda i: (0, 0)),
            pl.BlockSpec((D, D), lambda i: (0, 0)),
        ],
        out_specs=[
            pl.BlockSpec((blk, HW), lambda i: (i, 0)),
            pl.BlockSpec((blk, HW), lambda i: (i, 0)),
            pl.BlockSpec((blk, HW), lambda i: (i, 0)),
            pl.BlockSpec((blk, HW), lambda i: (i, 0)),
        ],
        out_shape=[
            jax.ShapeDtypeStruct((N, HW), jnp.float32),
            jax.ShapeDtypeStruct((N, HW), jnp.float32),
            jax.ShapeDtypeStruct((N, HW), jnp.float32),
            jax.ShapeDtypeStruct((N, HW), jnp.float32),
        ],
    )(x, rms_weight.reshape(1, D), W_down, W_l, W_r)


def _edge_kernel(xl0_hbm, xl1_hbm, xr0_hbm, xr1_hbm,
                 src_hbm, dstg_hbm, dsts_hbm, att_hbm,
                 acc_hbm,
                 idx_s, idx_g, idx_d, rows_l, rows_r, msg, att_v, acc_sh,
                 idx_s_b, idx_g_b, rows_l_b, rows_r_b, msg_b, idx_d_b,
                 idx_dp, idx_dp_b, sem, sem_i, sem_sc0, sem_sc1):
    c = lax.axis_index("c")
    s = lax.axis_index("s")

    def _zrow(i, _):
        for k in range(AW // NH):
            msg[i, pl.ds(k * NH, NH)] = jnp.zeros((NH,), jnp.float32)
        return _
    lax.fori_loop(0, B, _zrow, None)
    base_r = s * ROWS_PT
    nfull = ROWS_PT // B
    for k in range(nfull):
        pltpu.sync_copy(msg, acc_sh.at[pl.ds(base_r + k * B, B)])
    rem = ROWS_PT - nfull * B
    if rem:
        pltpu.sync_copy(msg.at[pl.ds(0, rem)],
                        acc_sh.at[pl.ds(base_r + nfull * B, rem)])

    pltpu.sync_copy(att_hbm, att_v)
    plsc.subcore_barrier()

    ebase = s * EPT

    msg2 = [msg, msg_b]
    sem_sc = [sem_sc0, sem_sc1]
    idx_d2 = [idx_d, idx_d_b]
    idx_s2 = [idx_s, idx_s_b]
    idx_g2 = [idx_g, idx_g_b]
    rows_l2 = [rows_l, rows_l_b]
    rows_r2 = [rows_r, rows_r_b]

    idx_dp2 = [idx_dp, idx_dp_b]

    def _issue_idx(b, buf):
        off = ebase + b * B
        pltpu.async_copy(src_hbm.at[pl.ds(off, B)], idx_s2[buf], sem_i)
        pltpu.async_copy(dstg_hbm.at[pl.ds(off, B)], idx_g2[buf], sem_i)
        pltpu.async_copy(dsts_hbm.at[pl.ds(off, B)], idx_dp2[buf], sem_i)

    def _wait_idx(buf):
        pltpu.make_async_copy(
            src_hbm.at[pl.ds(0, B)], idx_s2[buf], sem_i).wait()
        pltpu.make_async_copy(
            src_hbm.at[pl.ds(0, B)], idx_g2[buf], sem_i).wait()
        pltpu.make_async_copy(
            src_hbm.at[pl.ds(0, B)], idx_dp2[buf], sem_i).wait()

    def _issue_gathers(buf):
        @pl.when(c == 0)
        def _():
            pltpu.async_copy(xl0_hbm.at[idx_s2[buf]], rows_l2[buf], sem)
            pltpu.async_copy(xr0_hbm.at[idx_g2[buf]], rows_r2[buf], sem)

        @pl.when(c == 1)
        def _():
            pltpu.async_copy(xl1_hbm.at[idx_s2[buf]], rows_l2[buf], sem)
            pltpu.async_copy(xr1_hbm.at[idx_g2[buf]], rows_r2[buf], sem)

    _issue_idx(0, 0)
    _wait_idx(0)
    _issue_gathers(0)
    _issue_idx(1, 1)

    def _batch(k2, _):
        for ph in range(2):
            b = 2 * k2 + ph
            oth = 1 - ph
            rows_l_c = rows_l2[ph]
            rows_r_c = rows_r2[ph]

            @pl.when(b + 1 < NB)
            def _():
                _wait_idx(oth)
                _issue_gathers(oth)

            @pl.when(b >= 2)
            def _():
                pltpu.make_async_copy(
                    msg2[ph], acc_sh.at[idx_d2[ph]], sem_sc[ph]).wait()

            pltpu.make_async_copy(
                xl0_hbm.at[pl.ds(0, B)], rows_l_c, sem).wait()
            pltpu.make_async_copy(
                xl0_hbm.at[pl.ds(0, B)], rows_r_c, sem).wait()

            def _cpd(i, _):
                sl = pl.ds(i * NH, NH)
                idx_d2[ph][sl] = idx_dp2[ph][sl]
                return _
            lax.fori_loop(0, B // NH, _cpd, None)

            @pl.when(b + 2 < NB)
            def _():
                _issue_idx(b + 2, ph)

            _compute(rows_l_c, rows_r_c, msg2[ph])
            pltpu.async_copy(msg2[ph], acc_sh.at[idx_d2[ph]], sem_sc[ph],
                             add=True)
        return _

    def _compute(rows_l_c, rows_r_c, msg):
        @plsc.parallel_loop(0, B, unroll=5)
        def _edge(e):
            lch = [rows_l_c[e, pl.ds(j * NH, NH)] for j in range(HW // NH)]
            if True:
                zv = jnp.zeros((NH,), jnp.float32)
                slope = jnp.full((NH,), 0.2, jnp.float32)
                avs = []
                for h in range(2):
                    av = jnp.zeros((NH,), jnp.float32)
                    for j in (2 * h, 2 * h + 1):
                        z = lch[j] + rows_r_c[e, pl.ds(j * NH, NH)]
                        t = jnp.maximum(z, zv) + slope * jnp.minimum(z, zv)
                        av = av + t * att_v[pl.ds(c * HW + j * NH, NH)]
                    avs.append(av)
                lane = lax.broadcasted_iota(jnp.int32, (NH,), 0)

                dnums = lax.GatherDimensionNumbers(
                    offset_dims=(), collapsed_slice_dims=(0,),
                    start_index_map=(0,))

                def _g(v, idx):
                    return lax.gather(
                        v, idx[:, None], dnums, (1,),
                        mode=lax.GatherScatterMode.PROMISE_IN_BOUNDS)
                f0 = avs[0] + _g(avs[0], lane ^ 8)
                f1 = avs[1] + _g(avs[1], lane ^ 8)
                m = jnp.where(lane < 8, f0, f1)
                for sh in (4, 2, 1):
                    m = m + _g(m, lane ^ sh)
                w = jnp.exp(m)
                wb0 = _g(w, lane & 7)
                wb1 = _g(w, (lane & 7) | 8)
                msg[e, pl.ds(0 * NH, NH)] = wb0 * lch[0]
                msg[e, pl.ds(1 * NH, NH)] = wb0 * lch[1]
                msg[e, pl.ds(2 * NH, NH)] = wb1 * lch[2]
                msg[e, pl.ds(3 * NH, NH)] = wb1 * lch[3]
                msg[e, pl.ds(4 * NH, NH)] = w

    lax.fori_loop(0, NB // 2, _batch, None)
    for ph in range(2):
        pltpu.make_async_copy(msg2[ph], acc_sh.at[idx_d2[ph]],
                              sem_sc[ph]).wait()

    plsc.subcore_barrier()
    pltpu.sync_copy(acc_sh.at[pl.ds(base_r, ROWS_PT)],
                    acc_hbm.at[c, pl.ds(base_r, ROWS_PT)])


def _edge_pass(xl0, xl1, xr0, xr1, src, dstg, dsts, att_flat):
    mesh = plsc.VectorSubcoreMesh(core_axis_name="c", subcore_axis_name="s")
    k = functools.partial(
        pl.kernel,
        mesh=mesh,
        compiler_params=pltpu.CompilerParams(use_tc_tiling_on_sc=False),
        out_type=jax.ShapeDtypeStruct((NC, NPAD, AW), jnp.float32),
        scratch_types=[
            pltpu.VMEM((B,), jnp.int32),
            pltpu.VMEM((B,), jnp.int32),
            pltpu.VMEM((B,), jnp.int32),
            pltpu.VMEM((B, HW), jnp.float32),
            pltpu.VMEM((B, HW), jnp.float32),
            pltpu.VMEM((B, AW), jnp.float32),
            pltpu.VMEM((D,), jnp.float32),
            pltpu.VMEM_SHARED((NPAD, AW), jnp.float32),
            pltpu.VMEM((B,), jnp.int32),
            pltpu.VMEM((B,), jnp.int32),
            pltpu.VMEM((B, HW), jnp.float32),
            pltpu.VMEM((B, HW), jnp.float32),
            pltpu.VMEM((B, AW), jnp.float32),
            pltpu.VMEM((B,), jnp.int32),
            pltpu.VMEM((B,), jnp.int32),
            pltpu.VMEM((B,), jnp.int32),
            pltpu.SemaphoreType.DMA,
            pltpu.SemaphoreType.DMA,
            pltpu.SemaphoreType.DMA,
            pltpu.SemaphoreType.DMA,
        ],
    )(_edge_kernel)
    return k(xl0, xl1, xr0, xr1, src, dstg, dsts, att_flat)


def _final_body(a0_ref, a1_ref, wu_ref, out_ref):
    a0 = a0_ref[0]
    a1 = a1_ref[0]
    blk = a0.shape[0]
    smat = jnp.concatenate([a0[:, :HW], a1[:, :HW]], axis=1)
    den = jnp.concatenate(
        [jnp.broadcast_to(a0[:, HW:HW + 1], (blk, C)),
         jnp.broadcast_to(a0[:, HW + 8:HW + 9], (blk, C)),
         jnp.broadcast_to(a1[:, HW:HW + 1], (blk, C)),
         jnp.broadcast_to(a1[:, HW + 8:HW + 9], (blk, C))], axis=1)
    out_ref[...] = jnp.dot(smat / den, wu_ref[...],
                           preferred_element_type=jnp.float32)


def kernel(x, edge_index, rms_weight, W_down, W_l, W_r, att, W_up):
    xl0, xl1, xr0, xr1 = _projections(x, rms_weight, W_down, W_l, W_r)

    src0, dst0 = edge_index[0], edge_index[1]
    loop = jnp.arange(N, dtype=jnp.int32)
    pad = EP - E0 - N
    zpad = jnp.zeros((pad,), jnp.int32)
    src = jnp.concatenate([src0, loop, zpad])
    dsts = jnp.concatenate(
        [jnp.where(src0 != dst0, dst0, N), loop,
         jnp.full((pad,), N, jnp.int32)])
    dstg = jnp.concatenate([dst0, loop, zpad])

    acc = _edge_pass(xl0, xl1, xr0, xr1, src, dstg, dsts, att.reshape(-1))

    blk = 1000
    return pl.pallas_call(
        _final_body,
        grid=(N // blk,),
        in_specs=[
            pl.BlockSpec((1, blk, AW), lambda i: (0, i, 0)),
            pl.BlockSpec((1, blk, AW), lambda i: (1, i, 0)),
            pl.BlockSpec((D, D), lambda i: (0, 0)),
        ],
        out_specs=pl.BlockSpec((blk, D), lambda i: (i, 0)),
        out_shape=jax.ShapeDtypeStruct((N, D), jnp.float32),
    )(acc, acc, W_up)


# speedup vs baseline: 1.0692x; 1.0692x over previous
import functools

import jax
import jax.numpy as jnp
from jax import lax
from jax.experimental import pallas as pl
from jax.experimental.pallas import tpu as pltpu
from jax.experimental.pallas import tpu_sc as plsc

N = 10000
D = 128
H = 4
C = 32
NH = 16
NC = 2
NS = 16
HW = 64
B = 128
E0 = 320000
EP = 331776
EPT = EP // NS
NB = EPT // B
NPAD = 10016
ROWS_PT = NPAD // NS
AW = 80
def _proj_body(x_ref, w_ref, wd_ref, wl_ref, wr_ref, xl0_ref, xl1_ref,
               xr0_ref, xr1_ref):
    xb = x_ref[...]
    ms = jnp.mean(xb * xb, axis=1, keepdims=True)
    hb = xb * lax.rsqrt(ms + 1e-6) * w_ref[...]
    hb = jnp.dot(hb, wd_ref[...], preferred_element_type=jnp.float32)
    xl = jnp.dot(hb, wl_ref[...], preferred_element_type=jnp.float32)
    xr = jnp.dot(hb, wr_ref[...], preferred_element_type=jnp.float32)
    xl0_ref[...] = xl[:, :HW]
    xl1_ref[...] = xl[:, HW:]
    xr0_ref[...] = xr[:, :HW]
    xr1_ref[...] = xr[:, HW:]


def _projections(x, rms_weight, W_down, W_l, W_r):
    blk = 1000
    return pl.pallas_call(
        _proj_body,
        grid=(N // blk,),
        in_specs=[
            pl.BlockSpec((blk, D), lambda i: (i, 0)),
            pl.BlockSpec((1, D), lambda i: (0, 0)),
            pl.BlockSpec((D, D), lambda i: (0, 0)),
            pl.BlockSpec((D, D), lambda i: (0, 0)),
            pl.BlockSpec((D, D), lambda i: (0, 0)),
        ],
        out_specs=[
            pl.BlockSpec((blk, HW), lambda i: (i, 0)),
            pl.BlockSpec((blk, HW), lambda i: (i, 0)),
            pl.BlockSpec((blk, HW), lambda i: (i, 0)),
            pl.BlockSpec((blk, HW), lambda i: (i, 0)),
        ],
        out_shape=[
            jax.ShapeDtypeStruct((N, HW), jnp.float32),
            jax.ShapeDtypeStruct((N, HW), jnp.float32),
            jax.ShapeDtypeStruct((N, HW), jnp.float32),
            jax.ShapeDtypeStruct((N, HW), jnp.float32),
        ],
    )(x, rms_weight.reshape(1, D), W_down, W_l, W_r)


def _edge_kernel(xl0_hbm, xl1_hbm, xr0_hbm, xr1_hbm,
                 src_hbm, dstg_hbm, dsts_hbm, att_hbm,
                 acc_hbm,
                 idx_s, idx_g, idx_d, rows_l, rows_r, msg, att_v, acc_sh,
                 idx_s_b, idx_g_b, rows_l_b, rows_r_b, msg_b, idx_d_b,
                 idx_dp, idx_dp_b, sem, sem_i, sem_sc0, sem_sc1):
    c = lax.axis_index("c")
    s = lax.axis_index("s")

    def _zrow(i, _):
        for k in range(AW // NH):
            msg[i, pl.ds(k * NH, NH)] = jnp.zeros((NH,), jnp.float32)
        return _
    lax.fori_loop(0, B, _zrow, None)
    base_r = s * ROWS_PT
    nfull = ROWS_PT // B
    for k in range(nfull):
        pltpu.sync_copy(msg, acc_sh.at[pl.ds(base_r + k * B, B)])
    rem = ROWS_PT - nfull * B
    if rem:
        pltpu.sync_copy(msg.at[pl.ds(0, rem)],
                        acc_sh.at[pl.ds(base_r + nfull * B, rem)])

    pltpu.sync_copy(att_hbm, att_v)
    plsc.subcore_barrier()

    ebase = s * EPT

    msg2 = [msg, msg_b]
    sem_sc = [sem_sc0, sem_sc1]
    idx_d2 = [idx_d, idx_d_b]
    idx_s2 = [idx_s, idx_s_b]
    idx_g2 = [idx_g, idx_g_b]
    rows_l2 = [rows_l, rows_l_b]
    rows_r2 = [rows_r, rows_r_b]

    idx_dp2 = [idx_dp, idx_dp_b]

    def _issue_idx(b, buf):
        off = ebase + b * B
        pltpu.async_copy(src_hbm.at[pl.ds(off, B)], idx_s2[buf], sem_i)
        pltpu.async_copy(dstg_hbm.at[pl.ds(off, B)], idx_g2[buf], sem_i)
        pltpu.async_copy(dsts_hbm.at[pl.ds(off, B)], idx_dp2[buf], sem_i)

    def _wait_idx(buf):
        pltpu.make_async_copy(
            src_hbm.at[pl.ds(0, B)], idx_s2[buf], sem_i).wait()
        pltpu.make_async_copy(
            src_hbm.at[pl.ds(0, B)], idx_g2[buf], sem_i).wait()
        pltpu.make_async_copy(
            src_hbm.at[pl.ds(0, B)], idx_dp2[buf], sem_i).wait()

    def _issue_gathers(buf):
        @pl.when(c == 0)
        def _():
            pltpu.async_copy(xl0_hbm.at[idx_s2[buf]], rows_l2[buf], sem)
            pltpu.async_copy(xr0_hbm.at[idx_g2[buf]], rows_r2[buf], sem)

        @pl.when(c == 1)
        def _():
            pltpu.async_copy(xl1_hbm.at[idx_s2[buf]], rows_l2[buf], sem)
            pltpu.async_copy(xr1_hbm.at[idx_g2[buf]], rows_r2[buf], sem)

    _issue_idx(0, 0)
    _wait_idx(0)
    _issue_gathers(0)
    _issue_idx(1, 1)

    def _batch(k2, _):
        for ph in range(2):
            b = 2 * k2 + ph
            oth = 1 - ph
            rows_l_c = rows_l2[ph]
            rows_r_c = rows_r2[ph]

            @pl.when(b + 1 < NB)
            def _():
                _wait_idx(oth)
                _issue_gathers(oth)

            @pl.when(b >= 2)
            def _():
                pltpu.make_async_copy(
                    msg2[ph], acc_sh.at[idx_d2[ph]], sem_sc[ph]).wait()

            pltpu.make_async_copy(
                xl0_hbm.at[pl.ds(0, B)], rows_l_c, sem).wait()
            pltpu.make_async_copy(
                xl0_hbm.at[pl.ds(0, B)], rows_r_c, sem).wait()

            def _cpd(i, _):
                sl = pl.ds(i * NH, NH)
                idx_d2[ph][sl] = idx_dp2[ph][sl]
                return _
            lax.fori_loop(0, B // NH, _cpd, None)

            @pl.when(b + 2 < NB)
            def _():
                _issue_idx(b + 2, ph)

            _compute(rows_l_c, rows_r_c, msg2[ph])
            pltpu.async_copy(msg2[ph], acc_sh.at[idx_d2[ph]], sem_sc[ph],
                             add=True)
        return _

    def _compute(rows_l_c, rows_r_c, msg):
        @plsc.parallel_loop(0, B, unroll=4)
        def _edge(e):
            lch = [rows_l_c[e, pl.ds(j * NH, NH)]
                   for j in range(HW // NH)]
            zv = jnp.zeros((NH,), jnp.float32)
            slope = jnp.full((NH,), 0.2, jnp.float32)
            avs = []
            for h in range(2):
                    av = jnp.zeros((NH,), jnp.float32)
                    for j in (2 * h, 2 * h + 1):
                        z = lch[j] + rows_r_c[e, pl.ds(j * NH, NH)]
                        t = jnp.maximum(z, zv) + slope * jnp.minimum(z, zv)
                        av = av + t * att_v[pl.ds(c * HW + j * NH, NH)]
                    avs.append(av)
            lane = lax.broadcasted_iota(jnp.int32, (NH,), 0)

            dnums = lax.GatherDimensionNumbers(
                    offset_dims=(), collapsed_slice_dims=(0,),
                    start_index_map=(0,))

            def _g(v, idx):
                    return lax.gather(
                        v, idx[:, None], dnums, (1,),
                        mode=lax.GatherScatterMode.PROMISE_IN_BOUNDS)
            f0 = avs[0] + _g(avs[0], lane ^ 8)
            f1 = avs[1] + _g(avs[1], lane ^ 8)
            m = jnp.where(lane < 8, f0, f1)
            for sh in (4, 2, 1):
                    m = m + _g(m, lane ^ sh)
            w = jnp.exp(m)
            wb0 = _g(w, lane & 7)
            wb1 = _g(w, (lane & 7) | 8)
            msg[e, pl.ds(0 * NH, NH)] = wb0 * lch[0]
            msg[e, pl.ds(1 * NH, NH)] = wb0 * lch[1]
            msg[e, pl.ds(2 * NH, NH)] = wb1 * lch[2]
            msg[e, pl.ds(3 * NH, NH)] = wb1 * lch[3]
            msg[e, pl.ds(4 * NH, NH)] = w

    lax.fori_loop(0, NB // 2, _batch, None)
    for ph in range(2):
        pltpu.make_async_copy(msg2[ph], acc_sh.at[idx_d2[ph]],
                              sem_sc[ph]).wait()

    plsc.subcore_barrier()
    pltpu.sync_copy(acc_sh.at[pl.ds(base_r, ROWS_PT)],
                    acc_hbm.at[c, pl.ds(base_r, ROWS_PT)])


def _edge_pass(xl0, xl1, xr0, xr1, src, dstg, dsts, att_flat):
    mesh = plsc.VectorSubcoreMesh(core_axis_name="c", subcore_axis_name="s")
    k = functools.partial(
        pl.kernel,
        mesh=mesh,
        compiler_params=pltpu.CompilerParams(use_tc_tiling_on_sc=False),
        out_type=jax.ShapeDtypeStruct((NC, NPAD, AW), jnp.float32),
        scratch_types=[
            pltpu.VMEM((B,), jnp.int32),
            pltpu.VMEM((B,), jnp.int32),
            pltpu.VMEM((B,), jnp.int32),
            pltpu.VMEM((B, HW), jnp.float32),
            pltpu.VMEM((B, HW), jnp.float32),
            pltpu.VMEM((B, AW), jnp.float32),
            pltpu.VMEM((D,), jnp.float32),
            pltpu.VMEM_SHARED((NPAD, AW), jnp.float32),
            pltpu.VMEM((B,), jnp.int32),
            pltpu.VMEM((B,), jnp.int32),
            pltpu.VMEM((B, HW), jnp.float32),
            pltpu.VMEM((B, HW), jnp.float32),
            pltpu.VMEM((B, AW), jnp.float32),
            pltpu.VMEM((B,), jnp.int32),
            pltpu.VMEM((B,), jnp.int32),
            pltpu.VMEM((B,), jnp.int32),
            pltpu.SemaphoreType.DMA,
            pltpu.SemaphoreType.DMA,
            pltpu.SemaphoreType.DMA,
            pltpu.SemaphoreType.DMA,
        ],
    )(_edge_kernel)
    return k(xl0, xl1, xr0, xr1, src, dstg, dsts, att_flat)


def _final_body(a0_ref, a1_ref, wu_ref, out_ref):
    a0 = a0_ref[0]
    a1 = a1_ref[0]
    blk = a0.shape[0]
    smat = jnp.concatenate([a0[:, :HW], a1[:, :HW]], axis=1)
    den = jnp.concatenate(
        [jnp.broadcast_to(a0[:, HW:HW + 1], (blk, C)),
         jnp.broadcast_to(a0[:, HW + 8:HW + 9], (blk, C)),
         jnp.broadcast_to(a1[:, HW:HW + 1], (blk, C)),
         jnp.broadcast_to(a1[:, HW + 8:HW + 9], (blk, C))], axis=1)
    out_ref[...] = jnp.dot(smat / den, wu_ref[...],
                           preferred_element_type=jnp.float32)


def kernel(x, edge_index, rms_weight, W_down, W_l, W_r, att, W_up):
    xl0, xl1, xr0, xr1 = _projections(x, rms_weight, W_down, W_l, W_r)

    src0, dst0 = edge_index[0], edge_index[1]
    loop = jnp.arange(N, dtype=jnp.int32)
    pad = EP - E0 - N
    zpad = jnp.zeros((pad,), jnp.int32)
    src = jnp.concatenate([src0, loop, zpad])
    dsts = jnp.concatenate(
        [jnp.where(src0 != dst0, dst0, N), loop,
         jnp.full((pad,), N, jnp.int32)])
    dstg = jnp.concatenate([dst0, loop, zpad])

    acc = _edge_pass(xl0, xl1, xr0, xr1, src, dstg, dsts, att.reshape(-1))

    blk = 1000
    return pl.pallas_call(
        _final_body,
        grid=(N // blk,),
        in_specs=[
            pl.BlockSpec((1, blk, AW), lambda i: (0, i, 0)),
            pl.BlockSpec((1, blk, AW), lambda i: (1, i, 0)),
            pl.BlockSpec((D, D), lambda i: (0, 0)),
        ],
        out_specs=pl.BlockSpec((blk, D), lambda i: (i, 0)),
        out_shape=jax.ShapeDtypeStruct((N, D), jnp.float32),
    )(acc, acc, W_up)


# speedup vs baseline: 1.0697x; 1.0004x over previous
import functools

import jax
import jax.numpy as jnp
from jax import lax
from jax.experimental import pallas as pl
from jax.experimental.pallas import tpu as pltpu
from jax.experimental.pallas import tpu_sc as plsc

N = 10000
D = 128
H = 4
C = 32
NH = 16
NC = 2
NS = 16
HW = 64
B = 128
E0 = 320000
EP = 331776
EPT = EP // NS
NB = EPT // B
NPAD = 10016
ROWS_PT = NPAD // NS
AW = 80
def _proj_body(x_ref, w_ref, wd_ref, wl_ref, wr_ref, xl0_ref, xl1_ref,
               xr0_ref, xr1_ref):
    xb = x_ref[...]
    ms = jnp.mean(xb * xb, axis=1, keepdims=True)
    hb = xb * lax.rsqrt(ms + 1e-6) * w_ref[...]
    hb = jnp.dot(hb, wd_ref[...], preferred_element_type=jnp.float32)
    xl = jnp.dot(hb, wl_ref[...], preferred_element_type=jnp.float32)
    xr = jnp.dot(hb, wr_ref[...], preferred_element_type=jnp.float32)
    xl0_ref[...] = xl[:, :HW]
    xl1_ref[...] = xl[:, HW:]
    xr0_ref[...] = xr[:, :HW]
    xr1_ref[...] = xr[:, HW:]


def _projections(x, rms_weight, W_down, W_l, W_r):
    blk = 2000
    return pl.pallas_call(
        _proj_body,
        grid=(N // blk,),
        in_specs=[
            pl.BlockSpec((blk, D), lambda i: (i, 0)),
            pl.BlockSpec((1, D), lambda i: (0, 0)),
            pl.BlockSpec((D, D), lambda i: (0, 0)),
            pl.BlockSpec((D, D), lambda i: (0, 0)),
            pl.BlockSpec((D, D), lambda i: (0, 0)),
        ],
        out_specs=[
            pl.BlockSpec((blk, HW), lambda i: (i, 0)),
            pl.BlockSpec((blk, HW), lambda i: (i, 0)),
            pl.BlockSpec((blk, HW), lambda i: (i, 0)),
            pl.BlockSpec((blk, HW), lambda i: (i, 0)),
        ],
        out_shape=[
            jax.ShapeDtypeStruct((N, HW), jnp.float32),
            jax.ShapeDtypeStruct((N, HW), jnp.float32),
            jax.ShapeDtypeStruct((N, HW), jnp.float32),
            jax.ShapeDtypeStruct((N, HW), jnp.float32),
        ],
    )(x, rms_weight.reshape(1, D), W_down, W_l, W_r)


def _edge_kernel(xl0_hbm, xl1_hbm, xr0_hbm, xr1_hbm,
                 src_hbm, dstg_hbm, dsts_hbm, att_hbm,
                 acc_hbm,
                 idx_s, idx_g, idx_d, rows_l, rows_r, msg, att_v, acc_sh,
                 idx_s_b, idx_g_b, rows_l_b, rows_r_b, msg_b, idx_d_b,
                 idx_dp, idx_dp_b, sem, sem_i, sem_sc0, sem_sc1):
    c = lax.axis_index("c")
    s = lax.axis_index("s")

    def _zrow(i, _):
        for k in range(AW // NH):
            msg[i, pl.ds(k * NH, NH)] = jnp.zeros((NH,), jnp.float32)
        return _
    lax.fori_loop(0, B, _zrow, None)
    base_r = s * ROWS_PT
    nfull = ROWS_PT // B
    for k in range(nfull):
        pltpu.sync_copy(msg, acc_sh.at[pl.ds(base_r + k * B, B)])
    rem = ROWS_PT - nfull * B
    if rem:
        pltpu.sync_copy(msg.at[pl.ds(0, rem)],
                        acc_sh.at[pl.ds(base_r + nfull * B, rem)])

    pltpu.sync_copy(att_hbm, att_v)
    plsc.subcore_barrier()

    ebase = s * EPT

    msg2 = [msg, msg_b]
    sem_sc = [sem_sc0, sem_sc1]
    idx_d2 = [idx_d, idx_d_b]
    idx_s2 = [idx_s, idx_s_b]
    idx_g2 = [idx_g, idx_g_b]
    rows_l2 = [rows_l, rows_l_b]
    rows_r2 = [rows_r, rows_r_b]

    idx_dp2 = [idx_dp, idx_dp_b]

    def _issue_idx(b, buf):
        off = ebase + b * B
        pltpu.async_copy(src_hbm.at[pl.ds(off, B)], idx_s2[buf], sem_i)
        pltpu.async_copy(dstg_hbm.at[pl.ds(off, B)], idx_g2[buf], sem_i)
        pltpu.async_copy(dsts_hbm.at[pl.ds(off, B)], idx_dp2[buf], sem_i)

    def _wait_idx(buf):
        pltpu.make_async_copy(
            src_hbm.at[pl.ds(0, B)], idx_s2[buf], sem_i).wait()
        pltpu.make_async_copy(
            src_hbm.at[pl.ds(0, B)], idx_g2[buf], sem_i).wait()
        pltpu.make_async_copy(
            src_hbm.at[pl.ds(0, B)], idx_dp2[buf], sem_i).wait()

    def _issue_gathers(buf):
        @pl.when(c == 0)
        def _():
            pltpu.async_copy(xl0_hbm.at[idx_s2[buf]], rows_l2[buf], sem)
            pltpu.async_copy(xr0_hbm.at[idx_g2[buf]], rows_r2[buf], sem)

        @pl.when(c == 1)
        def _():
            pltpu.async_copy(xl1_hbm.at[idx_s2[buf]], rows_l2[buf], sem)
            pltpu.async_copy(xr1_hbm.at[idx_g2[buf]], rows_r2[buf], sem)

    _issue_idx(0, 0)
    _wait_idx(0)
    _issue_gathers(0)
    _issue_idx(1, 1)

    def _batch(k2, _):
        for ph in range(2):
            b = 2 * k2 + ph
            oth = 1 - ph
            rows_l_c = rows_l2[ph]
            rows_r_c = rows_r2[ph]

            @pl.when(b + 1 < NB)
            def _():
                _wait_idx(oth)
                _issue_gathers(oth)

            @pl.when(b >= 2)
            def _():
                pltpu.make_async_copy(
                    msg2[ph], acc_sh.at[idx_d2[ph]], sem_sc[ph]).wait()

            pltpu.make_async_copy(
                xl0_hbm.at[pl.ds(0, B)], rows_l_c, sem).wait()
            pltpu.make_async_copy(
                xl0_hbm.at[pl.ds(0, B)], rows_r_c, sem).wait()

            def _cpd(i, _):
                sl = pl.ds(i * NH, NH)
                idx_d2[ph][sl] = idx_dp2[ph][sl]
                return _
            lax.fori_loop(0, B // NH, _cpd, None)

            @pl.when(b + 2 < NB)
            def _():
                _issue_idx(b + 2, ph)

            _compute(rows_l_c, rows_r_c, msg2[ph])
            pltpu.async_copy(msg2[ph], acc_sh.at[idx_d2[ph]], sem_sc[ph],
                             add=True)
        return _

    def _compute(rows_l_c, rows_r_c, msg):
        @plsc.parallel_loop(0, B, unroll=4)
        def _edge(e):
            lch = [rows_l_c[e, pl.ds(j * NH, NH)]
                   for j in range(HW // NH)]
            zv = jnp.zeros((NH,), jnp.float32)
            slope = jnp.full((NH,), 0.2, jnp.float32)
            avs = []
            for h in range(2):
                    av = jnp.zeros((NH,), jnp.float32)
                    for j in (2 * h, 2 * h + 1):
                        z = lch[j] + rows_r_c[e, pl.ds(j * NH, NH)]
                        t = jnp.maximum(z, zv) + slope * jnp.minimum(z, zv)
                        av = av + t * att_v[pl.ds(c * HW + j * NH, NH)]
                    avs.append(av)
            lane = lax.broadcasted_iota(jnp.int32, (NH,), 0)

            dnums = lax.GatherDimensionNumbers(
                    offset_dims=(), collapsed_slice_dims=(0,),
                    start_index_map=(0,))

            def _g(v, idx):
                    return lax.gather(
                        v, idx[:, None], dnums, (1,),
                        mode=lax.GatherScatterMode.PROMISE_IN_BOUNDS)
            f0 = avs[0] + _g(avs[0], lane ^ 8)
            f1 = avs[1] + _g(avs[1], lane ^ 8)
            m = jnp.where(lane < 8, f0, f1)
            for sh in (4, 2, 1):
                    m = m + _g(m, lane ^ sh)
            w = jnp.exp(m)
            wb0 = _g(w, lane & 7)
            wb1 = _g(w, (lane & 7) | 8)
            msg[e, pl.ds(0 * NH, NH)] = wb0 * lch[0]
            msg[e, pl.ds(1 * NH, NH)] = wb0 * lch[1]
            msg[e, pl.ds(2 * NH, NH)] = wb1 * lch[2]
            msg[e, pl.ds(3 * NH, NH)] = wb1 * lch[3]
            msg[e, pl.ds(4 * NH, NH)] = w

    lax.fori_loop(0, NB // 2, _batch, None)
    for ph in range(2):
        pltpu.make_async_copy(msg2[ph], acc_sh.at[idx_d2[ph]],
                              sem_sc[ph]).wait()

    plsc.subcore_barrier()
    pltpu.sync_copy(acc_sh.at[pl.ds(base_r, ROWS_PT)],
                    acc_hbm.at[c, pl.ds(base_r, ROWS_PT)])


def _edge_pass(xl0, xl1, xr0, xr1, src, dstg, dsts, att_flat):
    mesh = plsc.VectorSubcoreMesh(core_axis_name="c", subcore_axis_name="s")
    k = functools.partial(
        pl.kernel,
        mesh=mesh,
        compiler_params=pltpu.CompilerParams(use_tc_tiling_on_sc=False),
        out_type=jax.ShapeDtypeStruct((NC, NPAD, AW), jnp.float32),
        scratch_types=[
            pltpu.VMEM((B,), jnp.int32),
            pltpu.VMEM((B,), jnp.int32),
            pltpu.VMEM((B,), jnp.int32),
            pltpu.VMEM((B, HW), jnp.float32),
            pltpu.VMEM((B, HW), jnp.float32),
            pltpu.VMEM((B, AW), jnp.float32),
            pltpu.VMEM((D,), jnp.float32),
            pltpu.VMEM_SHARED((NPAD, AW), jnp.float32),
            pltpu.VMEM((B,), jnp.int32),
            pltpu.VMEM((B,), jnp.int32),
            pltpu.VMEM((B, HW), jnp.float32),
            pltpu.VMEM((B, HW), jnp.float32),
            pltpu.VMEM((B, AW), jnp.float32),
            pltpu.VMEM((B,), jnp.int32),
            pltpu.VMEM((B,), jnp.int32),
            pltpu.VMEM((B,), jnp.int32),
            pltpu.SemaphoreType.DMA,
            pltpu.SemaphoreType.DMA,
            pltpu.SemaphoreType.DMA,
            pltpu.SemaphoreType.DMA,
        ],
    )(_edge_kernel)
    return k(xl0, xl1, xr0, xr1, src, dstg, dsts, att_flat)


def _final_body(a0_ref, a1_ref, wu_ref, out_ref):
    a0 = a0_ref[0]
    a1 = a1_ref[0]
    blk = a0.shape[0]
    smat = jnp.concatenate([a0[:, :HW], a1[:, :HW]], axis=1)
    den = jnp.concatenate(
        [jnp.broadcast_to(a0[:, HW:HW + 1], (blk, C)),
         jnp.broadcast_to(a0[:, HW + 8:HW + 9], (blk, C)),
         jnp.broadcast_to(a1[:, HW:HW + 1], (blk, C)),
         jnp.broadcast_to(a1[:, HW + 8:HW + 9], (blk, C))], axis=1)
    out_ref[...] = jnp.dot(smat / den, wu_ref[...],
                           preferred_element_type=jnp.float32)


def kernel(x, edge_index, rms_weight, W_down, W_l, W_r, att, W_up):
    xl0, xl1, xr0, xr1 = _projections(x, rms_weight, W_down, W_l, W_r)

    src0, dst0 = edge_index[0], edge_index[1]
    loop = jnp.arange(N, dtype=jnp.int32)
    pad = EP - E0 - N
    zpad = jnp.zeros((pad,), jnp.int32)
    src = jnp.concatenate([src0, loop, zpad])
    dsts = jnp.concatenate(
        [jnp.where(src0 != dst0, dst0, N), loop,
         jnp.full((pad,), N, jnp.int32)])
    dstg = jnp.concatenate([dst0, loop, zpad])

    acc = _edge_pass(xl0, xl1, xr0, xr1, src, dstg, dsts, att.reshape(-1))

    blk = 1000
    return pl.pallas_call(
        _final_body,
        grid=(N // blk,),
        in_specs=[
            pl.BlockSpec((1, blk, AW), lambda i: (0, i, 0)),
            pl.BlockSpec((1, blk, AW), lambda i: (1, i, 0)),
            pl.BlockSpec((D, D), lambda i: (0, 0)),
        ],
        out_specs=pl.BlockSpec((blk, D), lambda i: (i, 0)),
        out_shape=jax.ShapeDtypeStruct((N, D), jnp.float32),
    )(acc, acc, W_up)
